# baseline (speedup 1.0000x reference)
import jax
import jax.numpy as jnp
from jax import lax
from jax.experimental import pallas as pl
from jax.experimental.pallas import tpu as pltpu
from jax.experimental.pallas import tpu_sc as plsc

N_NODES = 10000
N_EDGES = 320000
H = 128

NC = 2
NS = 16
NW = NC * NS

NHALF = 4
EH = N_EDGES // NHALF
TOTAL_ROWS = 2 * EH
CHUNK = 128
ROWS_PER_W = 5120
PAD_ROWS = ROWS_PER_W * NW
N_CHUNKS = ROWS_PER_W // CHUNK
NBUF = 2


def _sc_gather_body(z_hbm, idx_hbm, out_hbm, z_sh, idx_v, rows_v,
                    sem_g0, sem_g1, sem_w0, sem_w1):
    sem_g = (sem_g0, sem_g1)
    sem_w = (sem_w0, sem_w1)
    sid = lax.axis_index("s")
    wid = sid * NC + lax.axis_index("c")
    row0 = wid * ROWS_PER_W

    @pl.when(sid == 0)
    def _():
        pltpu.sync_copy(z_hbm, z_sh)

    pltpu.sync_copy(idx_hbm.at[pl.ds(wid * N_CHUNKS, N_CHUNKS)], idx_v)
    plsc.subcore_barrier()

    @pl.loop(0, N_CHUNKS, step=NBUF)
    def _(g0):
        for b in range(NBUF):
            g = g0 + b

            @pl.when(g >= NBUF)
            def _():
                prev = g - NBUF
                pltpu.make_async_copy(
                    rows_v.at[b],
                    out_hbm.at[pl.ds(row0 + prev * CHUNK, CHUNK)],
                    sem_w[b]).wait()

            pltpu.async_copy(z_sh.at[idx_v.at[g]], rows_v.at[b], sem_g[b])
            pltpu.make_async_copy(z_hbm.at[pl.ds(0, CHUNK)],
                                  rows_v.at[b], sem_g[b]).wait()
            pltpu.async_copy(rows_v.at[b],
                             out_hbm.at[pl.ds(row0 + g * CHUNK, CHUNK)],
                             sem_w[b])

    for b in range(NBUF):
        last = N_CHUNKS - NBUF + b
        pltpu.make_async_copy(
            rows_v.at[b],
            out_hbm.at[pl.ds(row0 + last * CHUNK, CHUNK)],
            sem_w[b]).wait()


_sc_gather = pl.kernel(
    _sc_gather_body,
    out_type=jax.ShapeDtypeStruct((PAD_ROWS, H), jnp.float32),
    mesh=plsc.VectorSubcoreMesh(core_axis_name="c", subcore_axis_name="s",
                                num_cores=NC, num_subcores=NS),
    scratch_types=[
        pltpu.VMEM_SHARED((N_NODES, H), jnp.float32),
        pltpu.VMEM((N_CHUNKS, CHUNK), jnp.int32),
        pltpu.VMEM((NBUF, CHUNK, H), jnp.float32),
        pltpu.SemaphoreType.DMA,
        pltpu.SemaphoreType.DMA,
        pltpu.SemaphoreType.DMA,
        pltpu.SemaphoreType.DMA,
    ],
)

BE = 3200
NB = EH // BE
OUT_ROWS = BE // H


def _mlp_body(zs_ref, zd_ref, a_ref, b_ref, b1_ref, w2_ref, b2_ref, o_ref):
    zs = zs_ref[...]
    zd = zd_ref[...]
    avg = (zs + zd) * 0.5
    dif = zs - zd
    var = dif * dif
    dn = (((1,), (1,)), ((), ()))
    h1 = lax.dot_general(avg, a_ref[...], dn, preferred_element_type=jnp.float32)
    h1 = h1 + lax.dot_general(var, b_ref[...], dn, preferred_element_type=jnp.float32)
    h1 = jnp.maximum(h1 + b1_ref[...], 0.0)
    logit = jnp.sum(h1 * w2_ref[...], axis=1) + b2_ref[0, 0]
    o_ref[...] = jax.nn.sigmoid(logit).reshape(1, OUT_ROWS, H)


def _tc_mlp(gathered, a, b, b1, w2, b2):
    return pl.pallas_call(
        _mlp_body,
        grid=(NB,),
        in_specs=[
            pl.BlockSpec((BE, H), lambda i: (i, 0)),
            pl.BlockSpec((BE, H), lambda i: (i + NB, 0)),
            pl.BlockSpec((H, H), lambda i: (0, 0)),
            pl.BlockSpec((H, H), lambda i: (0, 0)),
            pl.BlockSpec((1, H), lambda i: (0, 0)),
            pl.BlockSpec((1, H), lambda i: (0, 0)),
            pl.BlockSpec((1, 1), lambda i: (0, 0), memory_space=pltpu.SMEM),
        ],
        out_specs=pl.BlockSpec((1, OUT_ROWS, H), lambda i: (i, 0, 0)),
        out_shape=jax.ShapeDtypeStruct((NB, OUT_ROWS, H), jnp.float32),
    )(gathered, gathered, a, b, b1, w2, b2)


def kernel(z, edge_index, W1_w, W1_b, W2_w, W2_b):
    ei = edge_index.astype(jnp.int32)
    pad = jnp.zeros((PAD_ROWS - TOTAL_ROWS,), jnp.int32)
    a = W1_w[:, :H]
    b = W1_w[:, H:]
    b1 = W1_b.reshape(1, H)
    w2 = W2_w.reshape(1, H)
    b2 = W2_b.reshape(1, 1)
    outs = []
    for h in range(NHALF):
        src = lax.slice_in_dim(ei[0], h * EH, (h + 1) * EH)
        dst = lax.slice_in_dim(ei[1], h * EH, (h + 1) * EH)
        idx_all = jnp.concatenate([src, dst, pad], axis=0)
        idx_2d = idx_all.reshape(PAD_ROWS // CHUNK, CHUNK)
        gathered = _sc_gather(z, idx_2d)
        outs.append(_tc_mlp(gathered, a, b, b1, w2, b2).reshape(EH))
    return jnp.concatenate(outs, axis=0)


# speedup vs baseline: 1.0524x; 1.0524x over previous
import jax
import jax.numpy as jnp
from jax import lax
from jax.experimental import pallas as pl
from jax.experimental.pallas import tpu as pltpu
from jax.experimental.pallas import tpu_sc as plsc

N_NODES = 10000
N_EDGES = 320000
H = 128

NC = 2
NS = 16
NW = NC * NS

EHS = (128000, 96000, 96000)
CHUNK = 128
NBUF = 2


def _make_sc_gather(n_chunks):
    rows_per_w = n_chunks * CHUNK
    pad_rows = rows_per_w * NW

    def body(z_hbm, idx_hbm, out_hbm, z_sh, idx_v, rows_v,
             sem_g0, sem_g1, sem_w0, sem_w1):
        sem_g = (sem_g0, sem_g1)
        sem_w = (sem_w0, sem_w1)
        sid = lax.axis_index("s")
        wid = sid * NC + lax.axis_index("c")
        row0 = wid * rows_per_w

        @pl.when(sid == 0)
        def _():
            pltpu.sync_copy(z_hbm, z_sh)

        pltpu.sync_copy(idx_hbm.at[pl.ds(wid * n_chunks, n_chunks)], idx_v)
        plsc.subcore_barrier()

        @pl.loop(0, n_chunks, step=NBUF)
        def _(g0):
            for b in range(NBUF):
                g = g0 + b

                @pl.when(g >= NBUF)
                def _():
                    prev = g - NBUF
                    pltpu.make_async_copy(
                        rows_v.at[b],
                        out_hbm.at[pl.ds(row0 + prev * CHUNK, CHUNK)],
                        sem_w[b]).wait()

                pltpu.async_copy(z_sh.at[idx_v.at[g]], rows_v.at[b],
                                 sem_g[b])
                pltpu.make_async_copy(z_hbm.at[pl.ds(0, CHUNK)],
                                      rows_v.at[b], sem_g[b]).wait()
                pltpu.async_copy(rows_v.at[b],
                                 out_hbm.at[pl.ds(row0 + g * CHUNK, CHUNK)],
                                 sem_w[b])

        for b in range(NBUF):
            last = n_chunks - NBUF + b
            pltpu.make_async_copy(
                rows_v.at[b],
                out_hbm.at[pl.ds(row0 + last * CHUNK, CHUNK)],
                sem_w[b]).wait()

    return pl.kernel(
        body,
        out_type=jax.ShapeDtypeStruct((pad_rows, H), jnp.float32),
        mesh=plsc.VectorSubcoreMesh(core_axis_name="c", subcore_axis_name="s",
                                    num_cores=NC, num_subcores=NS),
        scratch_types=[
            pltpu.VMEM_SHARED((N_NODES, H), jnp.float32),
            pltpu.VMEM((n_chunks, CHUNK), jnp.int32),
            pltpu.VMEM((NBUF, CHUNK, H), jnp.float32),
            pltpu.SemaphoreType.DMA,
            pltpu.SemaphoreType.DMA,
            pltpu.SemaphoreType.DMA,
            pltpu.SemaphoreType.DMA,
        ],
    )


BE = 6400
OUT_ROWS = BE // H


def _mlp_body(zs_ref, zd_ref, a_ref, b_ref, b1_ref, w2_ref, b2_ref, o_ref):
    zs = zs_ref[...]
    zd = zd_ref[...]
    avg = (zs + zd) * 0.5
    dif = zs - zd
    var = dif * dif
    dn = (((1,), (1,)), ((), ()))
    h1 = lax.dot_general(avg, a_ref[...], dn, preferred_element_type=jnp.float32)
    h1 = h1 + lax.dot_general(var, b_ref[...], dn, preferred_element_type=jnp.float32)
    h1 = jnp.maximum(h1 + b1_ref[...], 0.0)
    logit = jnp.sum(h1 * w2_ref[...], axis=1) + b2_ref[0, 0]
    o_ref[...] = jax.nn.sigmoid(logit).reshape(1, OUT_ROWS, H)


def _tc_mlp(gathered, a, b, b1, w2, b2, nb):
    return pl.pallas_call(
        _mlp_body,
        grid=(nb,),
        in_specs=[
            pl.BlockSpec((BE, H), lambda i: (i, 0)),
            pl.BlockSpec((BE, H), lambda i: (i + nb, 0)),
            pl.BlockSpec((H, H), lambda i: (0, 0)),
            pl.BlockSpec((H, H), lambda i: (0, 0)),
            pl.BlockSpec((1, H), lambda i: (0, 0)),
            pl.BlockSpec((1, H), lambda i: (0, 0)),
            pl.BlockSpec((1, 1), lambda i: (0, 0), memory_space=pltpu.SMEM),
        ],
        out_specs=pl.BlockSpec((1, OUT_ROWS, H), lambda i: (i, 0, 0)),
        out_shape=jax.ShapeDtypeStruct((nb, OUT_ROWS, H), jnp.float32),
    )(gathered, gathered, a, b, b1, w2, b2)


def _cdiv(x, d):
    return -(-x // d)


_sc_gathers = {}
for _eh in set(EHS):
    _nch = 8 * _cdiv(2 * _eh, NW * CHUNK * 8)
    _sc_gathers[_eh] = (_make_sc_gather(_nch), _nch)


def kernel(z, edge_index, W1_w, W1_b, W2_w, W2_b):
    ei = edge_index.astype(jnp.int32)
    a = W1_w[:, :H]
    b = W1_w[:, H:]
    b1 = W1_b.reshape(1, H)
    w2 = W2_w.reshape(1, H)
    b2 = W2_b.reshape(1, 1)
    outs = []
    e0 = 0
    for eh in EHS:
        gather_fn, n_chunks = _sc_gathers[eh]
        pad_rows = n_chunks * CHUNK * NW
        src = lax.slice_in_dim(ei[0], e0, e0 + eh)
        dst = lax.slice_in_dim(ei[1], e0, e0 + eh)
        pad = jnp.zeros((pad_rows - 2 * eh,), jnp.int32)
        idx_all = jnp.concatenate([src, dst, pad], axis=0)
        idx_2d = idx_all.reshape(pad_rows // CHUNK, CHUNK)
        gathered = gather_fn(z, idx_2d)
        outs.append(_tc_mlp(gathered, a, b, b1, w2, b2, eh // BE).reshape(eh))
        e0 += eh
    return jnp.concatenate(outs, axis=0)
